# baseline (speedup 1.0000x reference)
import functools

import jax
import jax.numpy as jnp
from jax import lax
from jax.experimental import pallas as pl
from jax.experimental.pallas import tpu as pltpu
from jax.experimental.pallas import tpu_sc as plsc

VOCAB = 100000
EMBED = 32
BATCH = 1024
SUPER = 128
PER_SUPER = SUPER // EMBED


_info = plsc.get_sparse_core_info()
_NC, _NS, _NL = _info.num_cores, _info.num_subcores, _info.num_lanes
_NW = _NC * _NS
_B_PER_W = BATCH // _NW


def _make_sc_gather():
    mesh = plsc.VectorSubcoreMesh(core_axis_name="c", subcore_axis_name="s")

    @functools.partial(
        pl.kernel,
        mesh=mesh,
        out_type=jax.ShapeDtypeStruct((BATCH, SUPER), jnp.float32),
        scratch_types=[
            pltpu.VMEM((_B_PER_W,), jnp.int32),
            pltpu.VMEM((_B_PER_W,), jnp.int32),
            pltpu.VMEM((_B_PER_W, SUPER), jnp.float32),
            pltpu.SemaphoreType.DMA,
        ],
    )
    def gather_kernel(table_hbm, idx_hbm, out_hbm, idx_v, sup_v, rows_v, sem):
        wid = lax.axis_index("s") * _NC + lax.axis_index("c")
        base = wid * _B_PER_W
        pltpu.sync_copy(idx_hbm.at[pl.ds(base, _B_PER_W)], idx_v)
        for g in range(_B_PER_W // _NL):
            sl = pl.ds(g * _NL, _NL)
            sup_v[sl] = lax.shift_right_logical(idx_v[sl], 2)
        pltpu.async_copy(table_hbm.at[sup_v], rows_v, sem).wait()
        pltpu.sync_copy(rows_v, out_hbm.at[pl.ds(base, _B_PER_W)])

    return gather_kernel


_sc_gather = _make_sc_gather()


_NV = 2048


def _proj_kernel(h128_ref, x_ref, w_ref, b_ref, out_ref):
    h128 = h128_ref[...]
    sub = x_ref[...] & (PER_SUPER - 1)
    h = jnp.zeros((BATCH, EMBED), jnp.float32)
    for s in range(PER_SUPER):
        h = jnp.where(sub == s, h128[:, s * EMBED:(s + 1) * EMBED], h)
    w = w_ref[...]
    acc = lax.dot_general(
        h, w,
        dimension_numbers=(((1,), (1,)), ((), ())),
        preferred_element_type=jnp.float32,
    )
    out_ref[...] = acc + b_ref[...]


def _projection(h128, x2d, W, b2d):
    grid = (pl.cdiv(VOCAB, _NV),)
    return pl.pallas_call(
        _proj_kernel,
        grid=grid,
        in_specs=[
            pl.BlockSpec((BATCH, SUPER), lambda j: (0, 0)),
            pl.BlockSpec((BATCH, 1), lambda j: (0, 0)),
            pl.BlockSpec((_NV, EMBED), lambda j: (j, 0)),
            pl.BlockSpec((1, _NV), lambda j: (0, j)),
        ],
        out_specs=pl.BlockSpec((BATCH, _NV), lambda j: (0, j)),
        out_shape=jax.ShapeDtypeStruct((BATCH, VOCAB), jnp.float32),
        compiler_params=pltpu.CompilerParams(
            dimension_semantics=("arbitrary",),
        ),
    )(h128, x2d, W, b2d)


def kernel(x, emb_table, W, b):
    table128 = emb_table.reshape(VOCAB // PER_SUPER, SUPER)
    h128 = _sc_gather(table128, x)
    return _projection(h128, x.reshape(BATCH, 1), W, b.reshape(1, VOCAB))


# speedup vs baseline: 1.0399x; 1.0399x over previous
import functools

import jax
import jax.numpy as jnp
from jax import lax
from jax.experimental import pallas as pl
from jax.experimental.pallas import tpu as pltpu
from jax.experimental.pallas import tpu_sc as plsc

VOCAB = 100000
EMBED = 32
BATCH = 1024


_info = plsc.get_sparse_core_info()
_NC, _NS, _NL = _info.num_cores, _info.num_subcores, _info.num_lanes
_NW = _NC * _NS
_B_PER_W = BATCH // _NW


def _make_sc_gather():
    mesh = plsc.VectorSubcoreMesh(core_axis_name="c", subcore_axis_name="s")

    @functools.partial(
        pl.kernel,
        mesh=mesh,
        out_type=jax.ShapeDtypeStruct((BATCH, EMBED), jnp.float32),
        scratch_types=[
            pltpu.VMEM((_B_PER_W,), jnp.int32),
            pltpu.VMEM((_B_PER_W, EMBED), jnp.float32),
            pltpu.SemaphoreType.DMA,
        ],
    )
    def gather_kernel(table_hbm, idx_hbm, out_hbm, idx_v, rows_v, sem):
        wid = lax.axis_index("s") * _NC + lax.axis_index("c")
        base = wid * _B_PER_W
        pltpu.sync_copy(idx_hbm.at[pl.ds(base, _B_PER_W)], idx_v)
        copies = []
        for j in range(_B_PER_W):
            row = idx_v[pl.ds((j // _NL) * _NL, _NL)][j % _NL]
            copies.append(
                pltpu.async_copy(
                    table_hbm.at[pl.ds(row, 1), :],
                    rows_v.at[pl.ds(j, 1), :],
                    sem,
                )
            )
        for c in copies:
            c.wait()
        pltpu.sync_copy(rows_v, out_hbm.at[pl.ds(base, _B_PER_W)])

    return gather_kernel


_sc_gather = _make_sc_gather()


_NV = 2048


def _proj_kernel(h_ref, w_ref, b_ref, out_ref):
    h = h_ref[...]
    w = w_ref[...]
    acc = lax.dot_general(
        h, w,
        dimension_numbers=(((1,), (1,)), ((), ())),
        preferred_element_type=jnp.float32,
    )
    out_ref[...] = acc + b_ref[...]


def _projection(h, W, b2d):
    grid = (pl.cdiv(VOCAB, _NV),)
    return pl.pallas_call(
        _proj_kernel,
        grid=grid,
        in_specs=[
            pl.BlockSpec((BATCH, EMBED), lambda j: (0, 0)),
            pl.BlockSpec((_NV, EMBED), lambda j: (j, 0)),
            pl.BlockSpec((1, _NV), lambda j: (0, j)),
        ],
        out_specs=pl.BlockSpec((BATCH, _NV), lambda j: (0, j)),
        out_shape=jax.ShapeDtypeStruct((BATCH, VOCAB), jnp.float32),
        compiler_params=pltpu.CompilerParams(
            dimension_semantics=("arbitrary",),
        ),
    )(h, W, b2d)


def kernel(x, emb_table, W, b):
    h = _sc_gather(emb_table, x)
    return _projection(h, W, b.reshape(1, VOCAB))


# speedup vs baseline: 3.3748x; 3.2453x over previous
import functools

import jax
import jax.numpy as jnp
from jax import lax
from jax.experimental import pallas as pl
from jax.experimental.pallas import tpu as pltpu
from jax.experimental.pallas import tpu_sc as plsc

VOCAB = 100000
EMBED = 32
BATCH = 1024


_info = plsc.get_sparse_core_info()
_NC, _NS, _NL = _info.num_cores, _info.num_subcores, _info.num_lanes
_NW = _NC * _NS
_B_PER_W = BATCH // _NW


def _make_sc_gather():
    mesh = plsc.VectorSubcoreMesh(core_axis_name="c", subcore_axis_name="s")

    @functools.partial(
        pl.kernel,
        mesh=mesh,
        out_type=jax.ShapeDtypeStruct((BATCH, EMBED), jnp.float32),
        scratch_types=[
            pltpu.VMEM((_B_PER_W,), jnp.int32),
            pltpu.VMEM((_B_PER_W, EMBED), jnp.float32),
            pltpu.SemaphoreType.DMA,
        ],
    )
    def gather_kernel(table_hbm, idx_hbm, out_hbm, idx_v, rows_v, sem):
        wid = lax.axis_index("s") * _NC + lax.axis_index("c")
        base = wid * _B_PER_W
        pltpu.sync_copy(idx_hbm.at[pl.ds(base, _B_PER_W)], idx_v)
        copies = []
        for j in range(_B_PER_W):
            row = idx_v[pl.ds((j // _NL) * _NL, _NL)][j % _NL]
            copies.append(
                pltpu.async_copy(
                    table_hbm.at[pl.ds(row, 1), :],
                    rows_v.at[pl.ds(j, 1), :],
                    sem,
                )
            )
        for c in copies:
            c.wait()
        pltpu.sync_copy(rows_v, out_hbm.at[pl.ds(base, _B_PER_W)])

    return gather_kernel


_sc_gather = _make_sc_gather()


_NV = 2048


def _proj_kernel(h_ref, wt_ref, b_ref, out_ref):
    h = h_ref[...]
    wt = wt_ref[...]
    acc = lax.dot_general(
        wt, h,
        dimension_numbers=(((0,), (1,)), ((), ())),
        preferred_element_type=jnp.float32,
    )
    bias = b_ref[...].reshape(_NV, 1)
    out_ref[...] = acc + bias


def _projection_t(h, Wt, b2d):
    grid = (pl.cdiv(VOCAB, _NV),)
    return pl.pallas_call(
        _proj_kernel,
        grid=grid,
        in_specs=[
            pl.BlockSpec((BATCH, EMBED), lambda j: (0, 0)),
            pl.BlockSpec((EMBED, _NV), lambda j: (0, j)),
            pl.BlockSpec((1, _NV), lambda j: (0, j)),
        ],
        out_specs=pl.BlockSpec((_NV, BATCH), lambda j: (j, 0)),
        out_shape=jax.ShapeDtypeStruct((VOCAB, BATCH), jnp.float32),
        compiler_params=pltpu.CompilerParams(
            dimension_semantics=("arbitrary",),
        ),
    )(h, Wt, b2d)


def kernel(x, emb_table, W, b):
    h = _sc_gather(emb_table, x)
    out_t = _projection_t(h, W.T, b.reshape(1, VOCAB))
    return out_t.T


# speedup vs baseline: 3.8823x; 1.1504x over previous
import functools

import jax
import jax.numpy as jnp
from jax import lax
from jax.experimental import pallas as pl
from jax.experimental.pallas import tpu as pltpu
from jax.experimental.pallas import tpu_sc as plsc

VOCAB = 100000
EMBED = 32
BATCH = 1024


_info = plsc.get_sparse_core_info()
_NC, _NS, _NL = _info.num_cores, _info.num_subcores, _info.num_lanes
_NW = _NC * _NS


def _make_sc_gather_t():
    mesh = plsc.VectorSubcoreMesh(core_axis_name="c", subcore_axis_name="s")

    @functools.partial(
        pl.kernel,
        mesh=mesh,
        compiler_params=pltpu.CompilerParams(needs_layout_passes=False),
        out_type=jax.ShapeDtypeStruct((EMBED, BATCH), jnp.float32),
        scratch_types=[
            pltpu.VMEM((VOCAB,), jnp.float32),
            pltpu.VMEM((BATCH,), jnp.int32),
            pltpu.VMEM((BATCH,), jnp.float32),
            pltpu.SemaphoreType.DMA,
        ],
    )
    def gather_kernel(et_hbm, idx_hbm, out_hbm, row_v, idx_v, hrow_v, sem):
        wid = lax.axis_index("s") * _NC + lax.axis_index("c")
        row_cp = pltpu.async_copy(et_hbm.at[wid], row_v, sem)
        pltpu.sync_copy(idx_hbm, idx_v)
        row_cp.wait()
        for g in range(BATCH // _NL):
            sl = pl.ds(g * _NL, _NL)
            hrow_v[sl] = plsc.load_gather(row_v, [idx_v[sl]])
        pltpu.sync_copy(hrow_v, out_hbm.at[wid])

    return gather_kernel


_sc_gather_t = _make_sc_gather_t()


_NV = 2048


def _proj_kernel(ht_ref, wt_ref, b_ref, out_ref):
    ht = ht_ref[...]
    wt = wt_ref[...]
    acc = lax.dot_general(
        wt, ht,
        dimension_numbers=(((0,), (0,)), ((), ())),
        preferred_element_type=jnp.float32,
    )
    bias = b_ref[...].reshape(_NV, 1)
    out_ref[...] = acc + bias


def _projection_t(ht, Wt, b2d):
    grid = (pl.cdiv(VOCAB, _NV),)
    return pl.pallas_call(
        _proj_kernel,
        grid=grid,
        in_specs=[
            pl.BlockSpec((EMBED, BATCH), lambda j: (0, 0)),
            pl.BlockSpec((EMBED, _NV), lambda j: (0, j)),
            pl.BlockSpec((1, _NV), lambda j: (0, j)),
        ],
        out_specs=pl.BlockSpec((_NV, BATCH), lambda j: (j, 0)),
        out_shape=jax.ShapeDtypeStruct((VOCAB, BATCH), jnp.float32),
        compiler_params=pltpu.CompilerParams(
            dimension_semantics=("arbitrary",),
        ),
    )(ht, Wt, b2d)


def kernel(x, emb_table, W, b):
    ht = _sc_gather_t(emb_table.T, x)
    out_t = _projection_t(ht, W.T, b.reshape(1, VOCAB))
    return out_t.T


# speedup vs baseline: 3.8900x; 1.0020x over previous
import functools

import jax
import jax.numpy as jnp
from jax import lax
from jax.experimental import pallas as pl
from jax.experimental.pallas import tpu as pltpu
from jax.experimental.pallas import tpu_sc as plsc

VOCAB = 100000
EMBED = 32
BATCH = 1024


_info = plsc.get_sparse_core_info()
_NC, _NS, _NL = _info.num_cores, _info.num_subcores, _info.num_lanes
_NW = _NC * _NS


def _make_sc_gather_t():
    mesh = plsc.VectorSubcoreMesh(core_axis_name="c", subcore_axis_name="s")

    @functools.partial(
        pl.kernel,
        mesh=mesh,
        compiler_params=pltpu.CompilerParams(needs_layout_passes=False),
        out_type=jax.ShapeDtypeStruct((EMBED, BATCH), jnp.float32),
        scratch_types=[
            pltpu.VMEM((VOCAB,), jnp.float32),
            pltpu.VMEM((BATCH,), jnp.int32),
            pltpu.VMEM((BATCH,), jnp.float32),
            pltpu.SemaphoreType.DMA,
        ],
    )
    def gather_kernel(et_hbm, idx_hbm, out_hbm, row_v, idx_v, hrow_v, sem):
        wid = lax.axis_index("s") * _NC + lax.axis_index("c")
        row_cp = pltpu.async_copy(et_hbm.at[wid], row_v, sem)
        pltpu.sync_copy(idx_hbm, idx_v)
        row_cp.wait()
        for g in range(BATCH // _NL):
            sl = pl.ds(g * _NL, _NL)
            hrow_v[sl] = plsc.load_gather(row_v, [idx_v[sl]])
        pltpu.sync_copy(hrow_v, out_hbm.at[wid])

    return gather_kernel


_sc_gather_t = _make_sc_gather_t()


_NV = 4096


def _proj_kernel(ht_ref, wt_ref, b_ref, out_ref):
    ht = ht_ref[...]
    wt = wt_ref[...]
    acc = lax.dot_general(
        wt, ht,
        dimension_numbers=(((0,), (0,)), ((), ())),
        preferred_element_type=jnp.float32,
    )
    bias = b_ref[...].reshape(_NV, 1)
    out_ref[...] = acc + bias


def _projection_t(ht, Wt, b2d):
    grid = (pl.cdiv(VOCAB, _NV),)
    return pl.pallas_call(
        _proj_kernel,
        grid=grid,
        in_specs=[
            pl.BlockSpec((EMBED, BATCH), lambda j: (0, 0)),
            pl.BlockSpec((EMBED, _NV), lambda j: (0, j)),
            pl.BlockSpec((1, _NV), lambda j: (0, j)),
        ],
        out_specs=pl.BlockSpec((_NV, BATCH), lambda j: (j, 0)),
        out_shape=jax.ShapeDtypeStruct((VOCAB, BATCH), jnp.float32),
        compiler_params=pltpu.CompilerParams(
            dimension_semantics=("arbitrary",),
        ),
    )(ht, Wt, b2d)


def kernel(x, emb_table, W, b):
    ht = _sc_gather_t(emb_table.T, x)
    out_t = _projection_t(ht, W.T, b.reshape(1, VOCAB))
    return out_t.T
